# Initial kernel scaffold; baseline (speedup 1.0000x reference)
import jax
import jax.numpy as jnp
from jax.experimental import pallas as pl


def kernel(feature, edge_index, W, b):
    raise NotImplementedError("write your pallas kernel here")


# speedup vs baseline: 1.3088x; 1.3088x over previous
import dataclasses

import jax
import jax.numpy as jnp
from jax import lax
from jax.experimental import pallas as pl
from jax.experimental.pallas import tpu as pltpu
from jax.experimental.pallas import tpu_sc as plsc

N = 10000
E = 160000
D = 256

NC = 2
NS = 16
NW = NC * NS
OWN = 320
ACC_R = 328
TRASH = OWN
SCE = 2000
NSUP = E // SCE
FIRE = 64
SBUF = 112
BTRASH = 96


def _agg_body(h_hbm, src_hbm, dst_hbm, z_hbm, agg_hbm,
              acc, srcst, dstst, srcbuf, locbuf, rows_v, spst, smem64, sem):
    c = lax.axis_index("c")
    s = lax.axis_index("s")
    wid = s * NC + c
    base = wid * OWN

    pltpu.sync_copy(z_hbm, acc)

    def _bounce_locs(bo, cnt):
        pltpu.sync_copy(locbuf.at[pl.ds(bo, cnt)],
                        spst.at[pl.ds(s * FIRE, cnt)])
        pltpu.sync_copy(spst.at[pl.ds(s * FIRE, cnt)],
                        smem64.at[pl.ds(0, cnt)])

    def _accum_rows(nrows):
        @pl.loop(0, nrows)
        def _row(r):
            rowoff = smem64[r] * D

            @pl.loop(0, D, step=16)
            def _col(j):
                plsc.addupdate(acc.at[pl.ds(rowoff + j, 16)],
                               rows_v[r, pl.ds(j, 16)])

    def _scan_step(i, fill):
        d = dstst[pl.ds(i * 16, 16)]
        sv = srcst[pl.ds(i * 16, 16)]
        m = (d >= base) & (d < base + OWN)
        loc = d - base
        pos = plsc.cumsum(m.astype(jnp.int32))
        idx = jnp.where(m, fill + pos - 1, BTRASH)
        plsc.store_scatter(srcbuf, [idx], sv)
        plsc.store_scatter(locbuf, [idx], loc)
        fill = fill + jnp.max(pos)

        def _fire(f):
            pltpu.async_copy(h_hbm.at[srcbuf.at[pl.ds(0, FIRE)]],
                             rows_v, sem).wait()
            _bounce_locs(0, FIRE)
            _accum_rows(FIRE)

            st = srcbuf[pl.ds(FIRE, 16)]
            lt = locbuf[pl.ds(FIRE, 16)]
            srcbuf[pl.ds(0, 16)] = st
            locbuf[pl.ds(0, 16)] = lt
            return f - FIRE

        return lax.cond(fill >= FIRE, _fire, lambda f: f, fill)

    def _super(sp, fill):
        off = sp * SCE
        pltpu.sync_copy(src_hbm.at[pl.ds(off, SCE)], srcst)
        pltpu.sync_copy(dst_hbm.at[pl.ds(off, SCE)], dstst)
        return lax.fori_loop(0, SCE // 16, _scan_step, fill)

    fill = lax.fori_loop(0, NSUP, _super, jnp.int32(0))

    pad_idx = fill + lax.iota(jnp.int32, 16)
    plsc.store_scatter(srcbuf, [pad_idx], lax.iota(jnp.int32, 16) * 8)
    plsc.store_scatter(locbuf, [pad_idx], jnp.full((16,), TRASH, jnp.int32))
    nbat = (fill + 15) // 16

    @pl.loop(0, nbat)
    def _tail(b):
        pltpu.async_copy(h_hbm.at[srcbuf.at[pl.ds(b * 16, 16)]],
                         rows_v.at[pl.ds(0, 16)], sem).wait()
        _bounce_locs(b * 16, 16)
        _accum_rows(16)

    @pl.when(wid < NW - 1)
    def _wb():
        pltpu.sync_copy(acc.at[pl.ds(0, OWN * D)],
                        agg_hbm.at[pl.ds(base * D, OWN * D)])

    @pl.when(wid == NW - 1)
    def _wb_last():
        pltpu.sync_copy(acc.at[pl.ds(0, (N - (NW - 1) * OWN) * D)],
                        agg_hbm.at[pl.ds(base * D, (N - (NW - 1) * OWN) * D)])


def _transpose_body(f_ref, h_ref):
    h_ref[...] = f_ref[...].T


def _linear_body(a_ref, w_ref, b_ref, o_ref):
    yt = lax.dot_general(w_ref[...], a_ref[...], (((1,), (1,)), ((), ())),
                         preferred_element_type=jnp.float32,
                         precision=lax.Precision.HIGHEST)
    o_ref[...] = jnp.maximum(yt + b_ref[...], 0.0)


def kernel(feature, edge_index, W, b):
    src = edge_index[0]
    dst = edge_index[1]
    zeros = jnp.zeros((ACC_R * D,), jnp.float32)

    BT = 512
    h = pl.pallas_call(
        _transpose_body,
        grid=(pl.cdiv(N, BT),),
        in_specs=[pl.BlockSpec((D, BT), lambda i: (0, i))],
        out_specs=pl.BlockSpec((BT, D), lambda i: (i, 0)),
        out_shape=jax.ShapeDtypeStruct((N, D), jnp.float32),
    )(feature)

    cp = pltpu.CompilerParams()
    if "needs_layout_passes" in pltpu.CompilerParams.__dataclass_fields__:
        cp = dataclasses.replace(cp, needs_layout_passes=False)
    agg1d = pl.kernel(
        _agg_body,
        out_type=jax.ShapeDtypeStruct((N * D,), jnp.float32),
        compiler_params=cp,
        mesh=plsc.VectorSubcoreMesh(core_axis_name="c", subcore_axis_name="s"),
        scratch_types=[
            pltpu.VMEM((ACC_R * D,), jnp.float32),
            pltpu.VMEM((SCE,), jnp.int32),
            pltpu.VMEM((SCE,), jnp.int32),
            pltpu.VMEM((SBUF,), jnp.int32),
            pltpu.VMEM((SBUF,), jnp.int32),
            pltpu.VMEM((FIRE, D), jnp.float32),
            pltpu.VMEM_SHARED((NS * FIRE,), jnp.int32),
            pltpu.SMEM((FIRE,), jnp.int32),
            pltpu.SemaphoreType.DMA,
        ],
    )(h, src, dst, zeros)
    agg = agg1d.reshape(N, D)

    BN = 512
    out = pl.pallas_call(
        _linear_body,
        grid=(pl.cdiv(N, BN),),
        in_specs=[pl.BlockSpec((BN, D), lambda i: (i, 0)),
                  pl.BlockSpec((D, D), lambda i: (0, 0)),
                  pl.BlockSpec((D, 1), lambda i: (0, 0))],
        out_specs=pl.BlockSpec((D, BN), lambda i: (0, i)),
        out_shape=jax.ShapeDtypeStruct((D, N), jnp.float32),
    )(agg, W, b.reshape(D, 1))

    return out
